# baseline (speedup 1.0000x reference)
import functools

import jax
import jax.numpy as jnp
from jax import lax
from jax.experimental import pallas as pl
from jax.experimental.pallas import tpu as pltpu
from jax.experimental.pallas import tpu_sc as plsc

EMBED_DIM = 64
BATCH = 16384
N_PROPS = 26
TABLE = 1000000

NUM_CORES = 2
NUM_SUBCORES = 16
NUM_WORKERS = NUM_CORES * NUM_SUBCORES
B_PER_W = BATCH // NUM_WORKERS
CHUNK = 128
N_CHUNKS = B_PER_W // CHUNK
LANES = 16

REPACK_W = 1024
REPACK_GRID = -(-TABLE // REPACK_W)


def _slice_body(xt_ref, idx_ref):
    idx_ref[...] = xt_ref[0, :]


def _repack_body(wt_ref, out_ref):
    y = wt_ref[...].T
    y3 = y.reshape(REPACK_W // 2, 2, EMBED_DIM)
    out_ref[...] = jnp.concatenate([y3[:, 0, :], y3[:, 1, :]], axis=1)


def _gather_body(idx_hbm, wp_hbm, out_hbm, idx_v, rows_v, sem):
    wid = lax.axis_index("s") * NUM_CORES + lax.axis_index("c")
    base = wid * B_PER_W

    for r in range(N_CHUNKS):
        pltpu.sync_copy(idx_hbm.at[pl.ds(base + r * CHUNK, CHUNK)], idx_v.at[r])
    for r in range(N_CHUNKS):
        for g in range(CHUNK // LANES):
            sl = pl.ds(g * LANES, LANES)
            idx_v[r, sl] = lax.shift_right_logical(idx_v[r, sl], 1)

    copies = [
        pltpu.async_copy(
            wp_hbm.at[idx_v.at[r]],
            rows_v.at[pl.ds(r * CHUNK, CHUNK)],
            sem,
        )
        for r in range(N_CHUNKS)
    ]
    for c in copies:
        c.wait()

    pltpu.sync_copy(rows_v, out_hbm.at[pl.ds(base, B_PER_W)])


def _extract_body(idx_ref, pairs_ref, out_ref):
    parity = (idx_ref[...] & 1)[:, None]
    left = pairs_ref[:, :EMBED_DIM]
    right = pairs_ref[:, EMBED_DIM:]
    out_ref[...] = jnp.where(parity == 1, right, left)


@jax.jit
def kernel(x, W):
    slice_col = pl.pallas_call(
        _slice_body,
        out_shape=jax.ShapeDtypeStruct((BATCH,), jnp.int32),
    )
    repack = pl.pallas_call(
        _repack_body,
        grid=(REPACK_GRID,),
        in_specs=[pl.BlockSpec((EMBED_DIM, REPACK_W), lambda b: (0, b))],
        out_specs=pl.BlockSpec((REPACK_W // 2, 2 * EMBED_DIM), lambda b: (b, 0)),
        out_shape=jax.ShapeDtypeStruct((TABLE // 2, 2 * EMBED_DIM), jnp.float32),
    )
    mesh = plsc.VectorSubcoreMesh(core_axis_name="c", subcore_axis_name="s")
    gather_pairs = functools.partial(
        pl.kernel,
        mesh=mesh,
        compiler_params=pltpu.CompilerParams(
            needs_layout_passes=False, use_tc_tiling_on_sc=False
        ),
        out_type=jax.ShapeDtypeStruct((BATCH, 2 * EMBED_DIM), jnp.float32),
        scratch_types=[
            pltpu.VMEM((N_CHUNKS, CHUNK), jnp.int32),
            pltpu.VMEM((B_PER_W, 2 * EMBED_DIM), jnp.float32),
            pltpu.SemaphoreType.DMA,
        ],
    )(_gather_body)
    extract = pl.pallas_call(
        _extract_body,
        out_shape=jax.ShapeDtypeStruct((BATCH, EMBED_DIM), jnp.float32),
    )

    idx = slice_col(x.T)
    wp = repack(W.T)
    pairs = gather_pairs(idx, wp)
    return extract(idx, pairs)


# speedup vs baseline: 1.6585x; 1.6585x over previous
import functools

import jax
import jax.numpy as jnp
from jax import lax
from jax.experimental import pallas as pl
from jax.experimental.pallas import tpu as pltpu
from jax.experimental.pallas import tpu_sc as plsc

EMBED_DIM = 64
BATCH = 16384
N_PROPS = 26
TABLE = 1000000

NUM_CORES = 2
NUM_SUBCORES = 16
NUM_WORKERS = NUM_CORES * NUM_SUBCORES
B_PER_W = BATCH // NUM_WORKERS
CHUNK = 128
N_CHUNKS = B_PER_W // CHUNK
LANES = 16

REPACK_W = 2048
REPACK_GRID = -(-TABLE // REPACK_W)
HALF_W = REPACK_W // 2


def _slice_body(xt_ref, idx_ref):
    idx_ref[...] = xt_ref[0, :]


def _repack_body(wt_ref, out_ref):
    y = wt_ref[...]
    left = y[:, :HALF_W].T
    right = y[:, HALF_W:].T
    out_ref[...] = jnp.concatenate([left, right], axis=1)


def _gather_body(idx_hbm, wp_hbm, out_hbm, idx_v, rows_v, sem):
    wid = lax.axis_index("s") * NUM_CORES + lax.axis_index("c")
    base = wid * B_PER_W

    for r in range(N_CHUNKS):
        pltpu.sync_copy(idx_hbm.at[pl.ds(base + r * CHUNK, CHUNK)], idx_v.at[r])
    for r in range(N_CHUNKS):
        for g in range(CHUNK // LANES):
            sl = pl.ds(g * LANES, LANES)
            i = idx_v[r, sl]
            idx_v[r, sl] = lax.shift_left(
                lax.shift_right_logical(i, 11), 10
            ) | (i & 1023)

    copies = [
        pltpu.async_copy(
            wp_hbm.at[idx_v.at[r]],
            rows_v.at[pl.ds(r * CHUNK, CHUNK)],
            sem,
        )
        for r in range(N_CHUNKS)
    ]
    for c in copies:
        c.wait()

    pltpu.sync_copy(rows_v, out_hbm.at[pl.ds(base, B_PER_W)])


def _extract_body(idx_ref, pairs_ref, out_ref):
    half = (lax.shift_right_logical(idx_ref[...], 10) & 1)[:, None]
    left = pairs_ref[:, :EMBED_DIM]
    right = pairs_ref[:, EMBED_DIM:]
    out_ref[...] = jnp.where(half == 1, right, left)


@jax.jit
def kernel(x, W):
    slice_col = pl.pallas_call(
        _slice_body,
        out_shape=jax.ShapeDtypeStruct((BATCH,), jnp.int32),
    )
    repack = pl.pallas_call(
        _repack_body,
        grid=(REPACK_GRID,),
        in_specs=[pl.BlockSpec((EMBED_DIM, REPACK_W), lambda b: (0, b))],
        out_specs=pl.BlockSpec((HALF_W, 2 * EMBED_DIM), lambda b: (b, 0)),
        out_shape=jax.ShapeDtypeStruct(
            (REPACK_GRID * HALF_W, 2 * EMBED_DIM), jnp.float32
        ),
    )
    mesh = plsc.VectorSubcoreMesh(core_axis_name="c", subcore_axis_name="s")
    gather_pairs = functools.partial(
        pl.kernel,
        mesh=mesh,
        compiler_params=pltpu.CompilerParams(
            needs_layout_passes=False, use_tc_tiling_on_sc=False
        ),
        out_type=jax.ShapeDtypeStruct((BATCH, 2 * EMBED_DIM), jnp.float32),
        scratch_types=[
            pltpu.VMEM((N_CHUNKS, CHUNK), jnp.int32),
            pltpu.VMEM((B_PER_W, 2 * EMBED_DIM), jnp.float32),
            pltpu.SemaphoreType.DMA,
        ],
    )(_gather_body)
    extract = pl.pallas_call(
        _extract_body,
        out_shape=jax.ShapeDtypeStruct((BATCH, EMBED_DIM), jnp.float32),
    )

    idx = slice_col(x.T)
    wp = repack(W.T)
    pairs = gather_pairs(idx, wp)
    return extract(idx, pairs)


# speedup vs baseline: 3.2042x; 1.9321x over previous
import functools

import jax
import jax.numpy as jnp
from jax import lax
from jax.experimental import pallas as pl
from jax.experimental.pallas import tpu as pltpu
from jax.experimental.pallas import tpu_sc as plsc

EMBED_DIM = 64
BATCH = 16384
N_PROPS = 26
TABLE = 1000000

NUM_CORES = 2
NUM_SUBCORES = 16
NUM_WORKERS = NUM_CORES * NUM_SUBCORES
B_PER_W = BATCH // NUM_WORKERS
CHUNK = 128
N_CHUNKS = B_PER_W // CHUNK
LANES = 16

REPACK_W = 32768
REPACK_GRID = -(-TABLE // REPACK_W)
HALF_W = REPACK_W // 2


def _slice_body(xt_ref, idx_ref):
    idx_ref[...] = xt_ref[0, :]


def _repack_body(wt_ref, out_ref):
    y = wt_ref[...]
    left = y[:, :HALF_W].T
    right = y[:, HALF_W:].T
    out_ref[...] = jnp.concatenate([left, right], axis=1)


def _gather_body(idx_hbm, wp_hbm, out_hbm, idx_v, rows_v, sem):
    wid = lax.axis_index("s") * NUM_CORES + lax.axis_index("c")
    base = wid * B_PER_W

    for r in range(N_CHUNKS):
        pltpu.sync_copy(idx_hbm.at[pl.ds(base + r * CHUNK, CHUNK)], idx_v.at[r])
    for r in range(N_CHUNKS):
        for g in range(CHUNK // LANES):
            sl = pl.ds(g * LANES, LANES)
            i = idx_v[r, sl]
            idx_v[r, sl] = lax.shift_left(
                lax.shift_right_logical(i, 15), 14
            ) | (i & 16383)

    copies = [
        pltpu.async_copy(
            wp_hbm.at[idx_v.at[r]],
            rows_v.at[pl.ds(r * CHUNK, CHUNK)],
            sem,
        )
        for r in range(N_CHUNKS)
    ]
    for c in copies:
        c.wait()

    pltpu.sync_copy(rows_v, out_hbm.at[pl.ds(base, B_PER_W)])


def _extract_body(idx_ref, pairs_ref, out_ref):
    half = (lax.shift_right_logical(idx_ref[...], 14) & 1)[:, None]
    left = pairs_ref[:, :EMBED_DIM]
    right = pairs_ref[:, EMBED_DIM:]
    out_ref[...] = jnp.where(half == 1, right, left)


@jax.jit
def kernel(x, W):
    slice_col = pl.pallas_call(
        _slice_body,
        out_shape=jax.ShapeDtypeStruct((BATCH,), jnp.int32),
    )
    repack = pl.pallas_call(
        _repack_body,
        grid=(REPACK_GRID,),
        in_specs=[pl.BlockSpec((EMBED_DIM, REPACK_W), lambda b: (0, b))],
        out_specs=pl.BlockSpec((HALF_W, 2 * EMBED_DIM), lambda b: (b, 0)),
        out_shape=jax.ShapeDtypeStruct(
            (REPACK_GRID * HALF_W, 2 * EMBED_DIM), jnp.float32
        ),
    )
    mesh = plsc.VectorSubcoreMesh(core_axis_name="c", subcore_axis_name="s")
    gather_pairs = functools.partial(
        pl.kernel,
        mesh=mesh,
        compiler_params=pltpu.CompilerParams(
            needs_layout_passes=False, use_tc_tiling_on_sc=False
        ),
        out_type=jax.ShapeDtypeStruct((BATCH, 2 * EMBED_DIM), jnp.float32),
        scratch_types=[
            pltpu.VMEM((N_CHUNKS, CHUNK), jnp.int32),
            pltpu.VMEM((B_PER_W, 2 * EMBED_DIM), jnp.float32),
            pltpu.SemaphoreType.DMA,
        ],
    )(_gather_body)
    eb = BATCH // 8
    extract = pl.pallas_call(
        _extract_body,
        grid=(8,),
        in_specs=[
            pl.BlockSpec((eb,), lambda b: (b,)),
            pl.BlockSpec((eb, 2 * EMBED_DIM), lambda b: (b, 0)),
        ],
        out_specs=pl.BlockSpec((eb, EMBED_DIM), lambda b: (b, 0)),
        out_shape=jax.ShapeDtypeStruct((BATCH, EMBED_DIM), jnp.float32),
    )

    idx = slice_col(x.T)
    wp = repack(W.T)
    pairs = gather_pairs(idx, wp)
    return extract(idx, pairs)
